# Initial kernel scaffold; baseline (speedup 1.0000x reference)
import jax
import jax.numpy as jnp
from jax.experimental import pallas as pl


def kernel(x, hx, cx, W_xt, W_tf, W_cf, W_tu, W_cu, W_th, W_ch, b_xt, b_tf, b_cf, b_tu, b_cu, b_th, b_ch):
    raise NotImplementedError("write your pallas kernel here")


# speedup vs baseline: 1.9123x; 1.9123x over previous
import jax
import jax.numpy as jnp
from jax.experimental import pallas as pl
from jax.experimental.pallas import tpu as pltpu

_B = 4096
_H = 1024
_BB = 256


def _dot_t(a, w):
    return jax.lax.dot_general(
        a, w, (((1,), (1,)), ((), ())), preferred_element_type=jnp.float32
    )


def _lstm_body(x_ref, hx_ref, cx_ref, wxt_ref, wtf_ref, wcf_ref, wtu_ref,
               wcu_ref, wth_ref, wch_ref, bxt_ref, btf_ref, bcf_ref, btu_ref,
               bcu_ref, bth_ref, bch_ref, hy_ref, cy_ref):
    x = x_ref[...]
    hx = hx_ref[...]
    cx = cx_ref[...]
    t = jnp.tanh(_dot_t(x, wxt_ref[...]) + bxt_ref[...]) + hx
    f = jax.nn.sigmoid(
        _dot_t(t, wtf_ref[...]) + _dot_t(cx, wcf_ref[...])
        + (btf_ref[...] + bcf_ref[...])
    )
    u = jax.nn.sigmoid(
        _dot_t(t, wtu_ref[...]) + _dot_t(cx, wcu_ref[...])
        + (btu_ref[...] + bcu_ref[...])
    ) * t
    cy = jnp.tanh(f * cx + u)
    hy = jnp.tanh(
        jax.nn.sigmoid(
            _dot_t(t, wth_ref[...]) + _dot_t(cy, wch_ref[...])
            + (bth_ref[...] + bch_ref[...])
        ) * cy
    )
    hy_ref[...] = hy
    cy_ref[...] = cy


def kernel(x, hx, cx, W_xt, W_tf, W_cf, W_tu, W_cu, W_th, W_ch,
           b_xt, b_tf, b_cf, b_tu, b_cu, b_th, b_ch):
    act_spec = pl.BlockSpec((_BB, _H), lambda i: (i, 0))
    w_spec = pl.BlockSpec((_H, _H), lambda i: (0, 0))
    b_spec = pl.BlockSpec((1, _H), lambda i: (0, 0))
    out = pl.pallas_call(
        _lstm_body,
        grid=(_B // _BB,),
        in_specs=[act_spec] * 3 + [w_spec] * 7 + [b_spec] * 7,
        out_specs=[
            pl.BlockSpec((_BB, _H), lambda i: (i, 0)),
            pl.BlockSpec((_BB, _H), lambda i: (i, 0)),
        ],
        out_shape=[
            jax.ShapeDtypeStruct((_B, _H), jnp.float32),
            jax.ShapeDtypeStruct((_B, _H), jnp.float32),
        ],
        compiler_params=pltpu.CompilerParams(
            dimension_semantics=("parallel",),
            vmem_limit_bytes=56 * 1024 * 1024,
        ),
        name="fused_lstm_cell",
    )(x, hx, cx, W_xt, W_tf, W_cf, W_tu, W_cu, W_th, W_ch,
      b_xt.reshape(1, _H), b_tf.reshape(1, _H), b_cf.reshape(1, _H),
      b_tu.reshape(1, _H), b_cu.reshape(1, _H), b_th.reshape(1, _H),
      b_ch.reshape(1, _H))
    return (out[0], out[1])
